# baseline (speedup 1.0000x reference)
import jax
import jax.numpy as jnp
from jax import lax
from jax.experimental import pallas as pl
from jax.experimental.pallas import tpu as pltpu
from jax.experimental.pallas import tpu_sc as plsc

_F = 128
_B = 8
_N = 5000
_ROW_W = _N * 5
_CHUNK = 1250
_VECS2 = 40
_ROW_PAD = 25216
_GRID = _F * _F


def _sc_scatter_body(preds_hbm, out_hbm, row_v, grid_v, sidx_a, sidx_b, sem):
    nc = 2
    wid = lax.axis_index("s") * nc + lax.axis_index("c")
    b = wid // 4
    q = wid % 4

    dma = pltpu.async_copy(preds_hbm.at[b], row_v, sem)

    zeros16 = jnp.zeros((16,), jnp.float32)

    def _zero(i, _):
        for u in range(8):
            grid_v[i, pl.ds(u * 16, 16)] = zeros16
        return 0

    lax.fori_loop(0, _F, _zero, 0)
    dma.wait()

    lanes = lax.broadcasted_iota(jnp.int32, (16,), 0)
    lane4 = lanes * 4
    group_masks = [(lanes >> 2) == g for g in range(4)]
    base_q = q * (_CHUNK * 5)

    def _half(base, boxid0, sidx_v):
        field = lanes * 5 + base
        c = plsc.load_gather(row_v, [field])
        x1 = plsc.load_gather(row_v, [field + 1])
        y1 = plsc.load_gather(row_v, [field + 2])
        x2 = plsc.load_gather(row_v, [field + 3])
        y2 = plsc.load_gather(row_v, [field + 4])

        ix1 = (x1 * _F).astype(jnp.int32)
        iy1 = (y1 * _F).astype(jnp.int32)
        ix2 = (x2 * _F).astype(jnp.int32)
        iy2 = (y2 * _F).astype(jnp.int32)

        in_range = (boxid0 + lanes) < _CHUNK
        valid = (ix2 > ix1) & (iy2 > iy1) & in_range
        cm = jnp.where(valid, c, 0.0)

        r1 = iy1 * _F
        r2 = iy2 * _F
        plsc.store_scatter(row_v, [field], cm)
        plsc.store_scatter(sidx_v, [lane4], r1 + ix1)
        plsc.store_scatter(sidx_v, [lane4 + 1], r1 + ix2)
        plsc.store_scatter(sidx_v, [lane4 + 2], r2 + ix1)
        plsc.store_scatter(sidx_v, [lane4 + 3], r2 + ix2)

        grp = lanes >> 2
        sign = jnp.where((lanes & 3) == 0, 1.0,
                         jnp.where((lanes & 3) == 3, 1.0, -1.0))
        for k in range(4):
            idxv = sidx_v[pl.ds(k * 16, 16)]
            cq = plsc.load_gather(row_v, [base + (k * 4 + grp) * 5])
            valv = cq * sign
            rowv = idxv >> 7
            colv = idxv & (_F - 1)
            for g in range(4):
                plsc.addupdate_scatter(grid_v, [rowv, colv], valv,
                                       mask=group_masks[g])

    def _step(i, _):
        base = base_q + i * 160
        _half(base, i * 32, sidx_a)
        _half(base + 80, i * 32 + 16, sidx_b)
        return 0

    lax.fori_loop(0, _VECS2, _step, 0)

    pltpu.sync_copy(grid_v, out_hbm.at[b, q])


def _sc_scatter(preds):
    mesh = plsc.VectorSubcoreMesh(core_axis_name="c", subcore_axis_name="s")
    return pl.kernel(
        _sc_scatter_body,
        out_type=jax.ShapeDtypeStruct((_B, 4, _F, _F), jnp.float32),
        mesh=mesh,
        scratch_types=[
            pltpu.VMEM((_ROW_PAD,), jnp.float32),
            pltpu.VMEM((_F, _F), jnp.float32),
            pltpu.VMEM((80,), jnp.int32),
            pltpu.VMEM((80,), jnp.int32),
            pltpu.SemaphoreType.DMA,
        ],
        compiler_params=pltpu.CompilerParams(needs_layout_passes=False),
    )(jnp.pad(preds.reshape(_B, _N * 5), ((0, 0), (0, _ROW_PAD - _ROW_W))))


def _tc_finish_kernel(g_ref, out_ref):
    row = lax.broadcasted_iota(jnp.int32, (_F, _F), 0)
    col = lax.broadcasted_iota(jnp.int32, (_F, _F), 1)
    tri = (col <= row).astype(jnp.float32)
    for b in range(_B):
        g = g_ref[b]
        grid = g[0] + g[1] + g[2] + g[3]
        cy = jax.lax.dot_general(tri, grid, (((1,), (0,)), ((), ())),
                                 preferred_element_type=jnp.float32)
        cxy = jax.lax.dot_general(cy, tri, (((1,), (1,)), ((), ())),
                                  preferred_element_type=jnp.float32)
        out_ref[b] = jax.nn.sigmoid(cxy)


def _tc_finish(partials):
    return pl.pallas_call(
        _tc_finish_kernel,
        out_shape=jax.ShapeDtypeStruct((_B, _F, _F), jnp.float32),
    )(partials)


def kernel(preds):
    return _tc_finish(_sc_scatter(preds))


# speedup vs baseline: 1.0004x; 1.0004x over previous
import jax
import jax.numpy as jnp
from jax import lax
from jax.experimental import pallas as pl
from jax.experimental.pallas import tpu as pltpu
from jax.experimental.pallas import tpu_sc as plsc

_F = 128
_B = 8
_N = 5000
_ROW_W = _N * 5
_CHUNK = 1250
_VECS2 = 40
_ROW_PAD = 25216
_GRID = _F * _F


def _sc_scatter_body(preds_hbm, out_hbm, row_v, grid_v, sidx_a, sidx_b, sem):
    nc = 2
    wid = lax.axis_index("s") * nc + lax.axis_index("c")
    b = wid // 4
    q = wid % 4

    dma = pltpu.async_copy(preds_hbm.at[b], row_v, sem)

    zeros16 = jnp.zeros((16,), jnp.float32)

    def _zero(i, _):
        for u in range(8):
            grid_v[i, pl.ds(u * 16, 16)] = zeros16
        return 0

    lax.fori_loop(0, _F, _zero, 0)
    dma.wait()

    lanes = lax.broadcasted_iota(jnp.int32, (16,), 0)
    lane4 = lanes * 4
    group_masks = [(lanes >> 2) == g for g in range(4)]
    base_q = q * (_CHUNK * 5)

    def _half(base, boxid0, sidx_v):
        field = lanes * 5 + base
        c = plsc.load_gather(row_v, [field])
        x1 = plsc.load_gather(row_v, [field + 1])
        y1 = plsc.load_gather(row_v, [field + 2])
        x2 = plsc.load_gather(row_v, [field + 3])
        y2 = plsc.load_gather(row_v, [field + 4])

        ix1 = (x1 * _F).astype(jnp.int32)
        iy1 = (y1 * _F).astype(jnp.int32)
        ix2 = (x2 * _F).astype(jnp.int32)
        iy2 = (y2 * _F).astype(jnp.int32)

        in_range = (boxid0 + lanes) < _CHUNK
        valid = (ix2 > ix1) & (iy2 > iy1) & in_range
        cm = jnp.where(valid, c, 0.0)

        r1 = iy1 * _F
        r2 = iy2 * _F
        plsc.store_scatter(row_v, [field], cm)
        plsc.store_scatter(sidx_v, [lane4], r1 + ix1)
        plsc.store_scatter(sidx_v, [lane4 + 1], r1 + ix2)
        plsc.store_scatter(sidx_v, [lane4 + 2], r2 + ix1)
        plsc.store_scatter(sidx_v, [lane4 + 3], r2 + ix2)

        grp = lanes >> 2
        sign = jnp.where((lanes & 3) == 0, 1.0,
                         jnp.where((lanes & 3) == 3, 1.0, -1.0))
        for k in range(4):
            idxv = sidx_v[pl.ds(k * 16, 16)]
            cq = plsc.load_gather(row_v, [base + (k * 4 + grp) * 5])
            valv = cq * sign
            rowv = idxv >> 7
            colv = idxv & (_F - 1)
            for g in range(4):
                plsc.addupdate_scatter(grid_v, [rowv, colv], valv,
                                       mask=group_masks[g])

    def _step(i, _):
        base = base_q + i * 160
        _half(base, i * 32, sidx_a)
        _half(base + 80, i * 32 + 16, sidx_b)
        return 0

    lax.fori_loop(0, _VECS2, _step, 0)

    pltpu.sync_copy(grid_v, out_hbm.at[b, q])


def _sc_scatter(preds):
    mesh = plsc.VectorSubcoreMesh(core_axis_name="c", subcore_axis_name="s")
    return pl.kernel(
        _sc_scatter_body,
        out_type=jax.ShapeDtypeStruct((_B, 4, _F, _F), jnp.float32),
        mesh=mesh,
        scratch_types=[
            pltpu.VMEM((_ROW_PAD,), jnp.float32),
            pltpu.VMEM((_F, _F), jnp.float32),
            pltpu.VMEM((80,), jnp.int32),
            pltpu.VMEM((80,), jnp.int32),
            pltpu.SemaphoreType.DMA,
        ],
        compiler_params=pltpu.CompilerParams(needs_layout_passes=False, skip_device_barrier=True),
    )(jnp.pad(preds.reshape(_B, _N * 5), ((0, 0), (0, _ROW_PAD - _ROW_W))))


def _tc_finish_kernel(g_ref, out_ref):
    row = lax.broadcasted_iota(jnp.int32, (_F, _F), 0)
    col = lax.broadcasted_iota(jnp.int32, (_F, _F), 1)
    tri = (col <= row).astype(jnp.float32)
    for b in range(_B):
        g = g_ref[b]
        grid = g[0] + g[1] + g[2] + g[3]
        cy = jax.lax.dot_general(tri, grid, (((1,), (0,)), ((), ())),
                                 preferred_element_type=jnp.float32)
        cxy = jax.lax.dot_general(cy, tri, (((1,), (1,)), ((), ())),
                                  preferred_element_type=jnp.float32)
        out_ref[b] = jax.nn.sigmoid(cxy)


def _tc_finish(partials):
    return pl.pallas_call(
        _tc_finish_kernel,
        out_shape=jax.ShapeDtypeStruct((_B, _F, _F), jnp.float32),
    )(partials)


def kernel(preds):
    return _tc_finish(_sc_scatter(preds))


# speedup vs baseline: 1.0340x; 1.0335x over previous
import jax
import jax.numpy as jnp
from jax import lax
from jax.experimental import pallas as pl
from jax.experimental.pallas import tpu as pltpu
from jax.experimental.pallas import tpu_sc as plsc

_F = 128
_B = 8
_N = 5000
_ROW_W = _N * 5
_CHUNK = 1250
_VECS2 = 40
_ROW_PAD = 25216
_GRID = _F * _F


def _sc_scatter_body(preds_hbm, out_hbm, row_v, grid_v, sidx_v4, sem):
    nc = 2
    wid = lax.axis_index("s") * nc + lax.axis_index("c")
    b = wid // 4
    q = wid % 4

    dma = pltpu.async_copy(preds_hbm.at[b], row_v, sem)

    zeros16 = jnp.zeros((16,), jnp.float32)

    def _zero(i, _):
        for u in range(8):
            grid_v[i, pl.ds(u * 16, 16)] = zeros16
        return 0

    lax.fori_loop(0, _F, _zero, 0)
    dma.wait()

    lanes = lax.broadcasted_iota(jnp.int32, (16,), 0)
    lane4 = lanes * 4
    group_masks = [(lanes >> 2) == g for g in range(4)]
    base_q = q * (_CHUNK * 5)

    def _half(base, boxid0, soff):
        field = lanes * 5 + base
        c = plsc.load_gather(row_v, [field])
        x1 = plsc.load_gather(row_v, [field + 1])
        y1 = plsc.load_gather(row_v, [field + 2])
        x2 = plsc.load_gather(row_v, [field + 3])
        y2 = plsc.load_gather(row_v, [field + 4])

        ix1 = (x1 * _F).astype(jnp.int32)
        iy1 = (y1 * _F).astype(jnp.int32)
        ix2 = (x2 * _F).astype(jnp.int32)
        iy2 = (y2 * _F).astype(jnp.int32)

        in_range = (boxid0 + lanes) < _CHUNK
        valid = (ix2 > ix1) & (iy2 > iy1) & in_range
        cm = jnp.where(valid, c, 0.0)

        r1 = iy1 * _F
        r2 = iy2 * _F
        plsc.store_scatter(row_v, [field], cm)
        plsc.store_scatter(sidx_v4, [soff + lane4], r1 + ix1)
        plsc.store_scatter(sidx_v4, [soff + lane4 + 1], r1 + ix2)
        plsc.store_scatter(sidx_v4, [soff + lane4 + 2], r2 + ix1)
        plsc.store_scatter(sidx_v4, [soff + lane4 + 3], r2 + ix2)

        grp = lanes >> 2
        sign = jnp.where((lanes & 3) == 0, 1.0,
                         jnp.where((lanes & 3) == 3, 1.0, -1.0))
        for k in range(4):
            idxv = sidx_v4[pl.ds(soff + k * 16, 16)]
            cq = plsc.load_gather(row_v, [base + (k * 4 + grp) * 5])
            valv = cq * sign
            rowv = idxv >> 7
            colv = idxv & (_F - 1)
            for g in range(4):
                plsc.addupdate_scatter(grid_v, [rowv, colv], valv,
                                       mask=group_masks[g])

    @plsc.parallel_loop(0, _VECS2, unroll=2)
    def _step(i):
        base = base_q + i * 160
        soff = (i & 1) * 160
        _half(base, i * 32, soff)
        _half(base + 80, i * 32 + 16, soff + 80)

    pltpu.sync_copy(grid_v, out_hbm.at[b, q])


def _sc_scatter(preds):
    mesh = plsc.VectorSubcoreMesh(core_axis_name="c", subcore_axis_name="s")
    return pl.kernel(
        _sc_scatter_body,
        out_type=jax.ShapeDtypeStruct((_B, 4, _F, _F), jnp.float32),
        mesh=mesh,
        scratch_types=[
            pltpu.VMEM((_ROW_PAD,), jnp.float32),
            pltpu.VMEM((_F, _F), jnp.float32),
            pltpu.VMEM((336,), jnp.int32),
            pltpu.SemaphoreType.DMA,
        ],
        compiler_params=pltpu.CompilerParams(needs_layout_passes=False),
    )(jnp.pad(preds.reshape(_B, _N * 5), ((0, 0), (0, _ROW_PAD - _ROW_W))))


def _tc_finish_kernel(g_ref, out_ref):
    row = lax.broadcasted_iota(jnp.int32, (_F, _F), 0)
    col = lax.broadcasted_iota(jnp.int32, (_F, _F), 1)
    tri = (col <= row).astype(jnp.float32)
    for b in range(_B):
        g = g_ref[b]
        grid = g[0] + g[1] + g[2] + g[3]
        cy = jax.lax.dot_general(tri, grid, (((1,), (0,)), ((), ())),
                                 preferred_element_type=jnp.float32)
        cxy = jax.lax.dot_general(cy, tri, (((1,), (1,)), ((), ())),
                                  preferred_element_type=jnp.float32)
        out_ref[b] = jax.nn.sigmoid(cxy)


def _tc_finish(partials):
    return pl.pallas_call(
        _tc_finish_kernel,
        out_shape=jax.ShapeDtypeStruct((_B, _F, _F), jnp.float32),
    )(partials)


def kernel(preds):
    return _tc_finish(_sc_scatter(preds))
